# baseline (speedup 1.0000x reference)
import jax
import jax.numpy as jnp
from jax.experimental import pallas as pl
from jax.experimental.pallas import tpu as pltpu

_N = 10000
_BI = 200
_BA = 2000


def _down_body(x_ref, wdt_ref, bd_ref, w1_ref, b1_ref, a_ref):
    d = jnp.dot(x_ref[...], wdt_ref[...], preferred_element_type=jnp.float32)
    d = d + bd_ref[...]
    a = jnp.dot(d, w1_ref[...], preferred_element_type=jnp.float32)
    a_ref[...] = a + b1_ref[...]


def _prop1_body(h_ref, a_ref, w2_ref, b2_ref, o_ref):
    t = jnp.dot(h_ref[...], a_ref[...], preferred_element_type=jnp.float32)
    t = jnp.maximum(t, 0.0)
    o_ref[...] = jnp.dot(t, w2_ref[...], preferred_element_type=jnp.float32) + b2_ref[...]


def _prop2_body(h_ref, b_ref, x_ref, wut_ref, bu_ref, wct_ref, bc_ref, o_ref):
    t = jnp.dot(h_ref[...], b_ref[...], preferred_element_type=jnp.float32)
    up = jnp.dot(t, wut_ref[...], preferred_element_type=jnp.float32) + bu_ref[...]
    enh = x_ref[...] + up
    o_ref[...] = jnp.dot(enh, wct_ref[...], preferred_element_type=jnp.float32) + bc_ref[...]


def _full(shape):
    return pl.BlockSpec(shape, lambda i: (0, 0))


def kernel(combined_features, H, Wd, bd, W1, b1, W2, b2, Wu, bu, Wc, bc):
    x = combined_features
    n, in_dim = x.shape
    hid = W1.shape[0]
    nc = Wc.shape[0]

    wdt = Wd.T
    wut = Wu.T
    wct = Wc.T
    bd2 = bd.reshape(1, -1)
    b12 = b1.reshape(1, -1)
    b22 = b2.reshape(1, -1)
    bu2 = bu.reshape(1, -1)
    bc2 = bc.reshape(1, -1)

    params = pltpu.CompilerParams(dimension_semantics=("parallel",))

    a = pl.pallas_call(
        _down_body,
        grid=(n // _BA,),
        in_specs=[
            pl.BlockSpec((_BA, in_dim), lambda i: (i, 0)),
            _full((in_dim, hid)),
            _full((1, hid)),
            _full((hid, hid)),
            _full((1, hid)),
        ],
        out_specs=pl.BlockSpec((_BA, hid), lambda i: (i, 0)),
        out_shape=jax.ShapeDtypeStruct((n, hid), jnp.float32),
        compiler_params=params,
    )(x, wdt, bd2, W1, b12)

    b = pl.pallas_call(
        _prop1_body,
        grid=(n // _BI,),
        in_specs=[
            pl.BlockSpec((_BI, n), lambda i: (i, 0)),
            _full((n, hid)),
            _full((hid, hid)),
            _full((1, hid)),
        ],
        out_specs=pl.BlockSpec((_BI, hid), lambda i: (i, 0)),
        out_shape=jax.ShapeDtypeStruct((n, hid), jnp.float32),
        compiler_params=params,
    )(H, a, W2, b22)

    out = pl.pallas_call(
        _prop2_body,
        grid=(n // _BI,),
        in_specs=[
            pl.BlockSpec((_BI, n), lambda i: (i, 0)),
            _full((n, hid)),
            pl.BlockSpec((_BI, in_dim), lambda i: (i, 0)),
            _full((hid, in_dim)),
            _full((1, in_dim)),
            _full((in_dim, nc)),
            _full((1, nc)),
        ],
        out_specs=pl.BlockSpec((_BI, nc), lambda i: (i, 0)),
        out_shape=jax.ShapeDtypeStruct((n, nc), jnp.float32),
        compiler_params=params,
    )(H, b, x, wut, bu2, wct, bc2)

    return out
